# baseline (device time: 20365 ns/iter reference)
import jax
import jax.numpy as jnp
from jax import lax
from jax.experimental import pallas as pl
from jax.experimental.pallas import tpu as pltpu

CH = 32
NX = 10
NZ = 6
NCH = NX + NZ
YLAG = 3


def kernel(partial, resid, gamma):
    m, d = resid.shape
    half = m // 2
    assert NCH * CH == half
    p = partial.reshape(m, d).astype(jnp.bfloat16)
    g = gamma.reshape(1, d)

    def body(
        p_ref, resid_ref, g_ref, out_ref,
        prbuf,
        x_ssem, x_rsem, z1_ssem, z1_rsem, y_ssem, y_rsem, z2_ssem, z2_rsem,
    ):
        my_x = lax.axis_index("x")
        my_y = lax.axis_index("y")
        my_z = lax.axis_index("z")
        xn = (1 - my_x, my_y, my_z)
        yn = (my_x, 1 - my_y, my_z)
        zn = (my_x, my_y, my_z ^ 1)
        h = my_x ^ my_y
        zp = my_z & 1
        my0 = h * half
        ot0 = (1 - h) * half

        def rx(i):
            return jnp.where(
                zp == 0, CH * i, CH * ((NX + i) if i < NZ else i)
            )

        def rz(j):
            return jnp.where(zp == 0, CH * (NX + j), CH * j)

        barrier_sem = pltpu.get_barrier_semaphore()
        for nbr in (xn, yn, zn):
            pl.semaphore_signal(
                barrier_sem, inc=1, device_id=nbr,
                device_id_type=pl.DeviceIdType.MESH,
            )
        pl.semaphore_wait(barrier_sem, 3)

        x_rdmas = []
        for i in range(NX):
            rd = pltpu.make_async_remote_copy(
                src_ref=p_ref.at[pl.ds(ot0 + rx(i), CH), :],
                dst_ref=prbuf.at[i],
                send_sem=x_ssem.at[i],
                recv_sem=x_rsem.at[i],
                device_id=xn,
                device_id_type=pl.DeviceIdType.MESH,
            )
            rd.start()
            x_rdmas.append(rd)

        z1_rdmas = []
        y_rdmas = []
        z2_rdmas = []
        for c in range(NCH):
            if c < NX:
                x_rdmas[c].wait_recv()
                if c < NZ:
                    rd = pltpu.make_async_remote_copy(
                        src_ref=prbuf.at[c],
                        dst_ref=prbuf.at[NX + c],
                        send_sem=z1_ssem.at[c],
                        recv_sem=z1_rsem.at[c],
                        device_id=zn,
                        device_id_type=pl.DeviceIdType.MESH,
                    )
                    rd.start()
                    z1_rdmas.append(rd)
                rr = rx(c)
            else:
                z1_rdmas[c - NX].wait_recv()
                rr = rz(c - NX)
            r0 = my0 + rr
            yv = (p_ref[pl.ds(r0, CH), :] + prbuf[c]).astype(
                jnp.float32
            ) + resid_ref[pl.ds(r0, CH), :]
            inv = lax.rsqrt(
                jnp.mean(yv * yv, axis=-1, keepdims=True) + 1e-6
            )
            out_ref[pl.ds(r0, CH), :] = ((yv * inv) * g_ref[...]).astype(
                jnp.bfloat16
            )
            if c < NX:
                rd = pltpu.make_async_remote_copy(
                    src_ref=out_ref.at[pl.ds(r0, CH), :],
                    dst_ref=out_ref.at[pl.ds(r0, CH), :],
                    send_sem=y_ssem.at[c],
                    recv_sem=y_rsem.at[c],
                    device_id=yn,
                    device_id_type=pl.DeviceIdType.MESH,
                )
                rd.start()
                y_rdmas.append(rd)
            j = c - YLAG
            if 0 <= j < NZ:
                y_rdmas[j].wait_recv()
                rd = pltpu.make_async_remote_copy(
                    src_ref=out_ref.at[pl.ds(ot0 + rx(j), CH), :],
                    dst_ref=out_ref.at[pl.ds(ot0 + rx(j), CH), :],
                    send_sem=z2_ssem.at[j],
                    recv_sem=z2_rsem.at[j],
                    device_id=zn,
                    device_id_type=pl.DeviceIdType.MESH,
                )
                rd.start()
                z2_rdmas.append(rd)

        for i in range(NZ, NX):
            y_rdmas[i].wait_recv()
        for j in range(NZ):
            z2_rdmas[j].wait_recv()

        for i in range(NX):
            x_rdmas[i].wait_send()
            y_rdmas[i].wait_send()
        for j in range(NZ):
            z1_rdmas[j].wait_send()
            z2_rdmas[j].wait_send()

    return pl.pallas_call(
        body,
        out_shape=jax.ShapeDtypeStruct((m, d), jnp.bfloat16),
        in_specs=[
            pl.BlockSpec(memory_space=pltpu.VMEM),
            pl.BlockSpec(memory_space=pltpu.VMEM),
            pl.BlockSpec(memory_space=pltpu.VMEM),
        ],
        out_specs=pl.BlockSpec(memory_space=pltpu.VMEM),
        scratch_shapes=[
            pltpu.VMEM((NCH, CH, d), jnp.bfloat16),
            pltpu.SemaphoreType.DMA((NX,)),
            pltpu.SemaphoreType.DMA((NX,)),
            pltpu.SemaphoreType.DMA((NZ,)),
            pltpu.SemaphoreType.DMA((NZ,)),
            pltpu.SemaphoreType.DMA((NX,)),
            pltpu.SemaphoreType.DMA((NX,)),
            pltpu.SemaphoreType.DMA((NZ,)),
            pltpu.SemaphoreType.DMA((NZ,)),
        ],
        compiler_params=pltpu.CompilerParams(collective_id=0),
    )(p, resid, g)


# device time: 19651 ns/iter; 1.0363x vs baseline; 1.0363x over previous
import jax
import jax.numpy as jnp
from jax import lax
from jax.experimental import pallas as pl
from jax.experimental.pallas import tpu as pltpu

CH = 32
NX = 11
NZ = 5
NCH = NX + NZ
YLAG = 3


def kernel(partial, resid, gamma):
    m, d = resid.shape
    half = m // 2
    assert NCH * CH == half
    p = partial.reshape(m, d).astype(jnp.bfloat16)
    g = gamma.reshape(1, d)

    def body(
        p_ref, resid_ref, g_ref, out_ref,
        prbuf,
        x_ssem, x_rsem, z1_ssem, z1_rsem, y_ssem, y_rsem, z2_ssem, z2_rsem,
    ):
        my_x = lax.axis_index("x")
        my_y = lax.axis_index("y")
        my_z = lax.axis_index("z")
        xn = (1 - my_x, my_y, my_z)
        yn = (my_x, 1 - my_y, my_z)
        zn = (my_x, my_y, my_z ^ 1)
        h = my_x ^ my_y
        zp = my_z & 1
        my0 = h * half
        ot0 = (1 - h) * half

        def rx(i):
            return jnp.where(
                zp == 0, CH * i, CH * ((NX + i) if i < NZ else i)
            )

        def rz(j):
            return jnp.where(zp == 0, CH * (NX + j), CH * j)

        barrier_sem = pltpu.get_barrier_semaphore()
        for nbr in (xn, yn, zn):
            pl.semaphore_signal(
                barrier_sem, inc=1, device_id=nbr,
                device_id_type=pl.DeviceIdType.MESH,
            )
        pl.semaphore_wait(barrier_sem, 3)

        x_rdmas = []
        for i in range(NX):
            rd = pltpu.make_async_remote_copy(
                src_ref=p_ref.at[pl.ds(ot0 + rx(i), CH), :],
                dst_ref=prbuf.at[i],
                send_sem=x_ssem.at[i],
                recv_sem=x_rsem.at[i],
                device_id=xn,
                device_id_type=pl.DeviceIdType.MESH,
            )
            rd.start()
            x_rdmas.append(rd)

        z1_rdmas = []
        y_rdmas = []
        z2_rdmas = []
        for c in range(NCH):
            if c < NX:
                x_rdmas[c].wait_recv()
                if c < NZ:
                    rd = pltpu.make_async_remote_copy(
                        src_ref=prbuf.at[c],
                        dst_ref=prbuf.at[NX + c],
                        send_sem=z1_ssem.at[c],
                        recv_sem=z1_rsem.at[c],
                        device_id=zn,
                        device_id_type=pl.DeviceIdType.MESH,
                    )
                    rd.start()
                    z1_rdmas.append(rd)
                rr = rx(c)
            else:
                z1_rdmas[c - NX].wait_recv()
                rr = rz(c - NX)
            r0 = my0 + rr
            yv = (p_ref[pl.ds(r0, CH), :] + prbuf[c]).astype(
                jnp.float32
            ) + resid_ref[pl.ds(r0, CH), :]
            inv = lax.rsqrt(
                jnp.mean(yv * yv, axis=-1, keepdims=True) + 1e-6
            )
            out_ref[pl.ds(r0, CH), :] = ((yv * inv) * g_ref[...]).astype(
                jnp.bfloat16
            )
            if c < NX:
                rd = pltpu.make_async_remote_copy(
                    src_ref=out_ref.at[pl.ds(r0, CH), :],
                    dst_ref=out_ref.at[pl.ds(r0, CH), :],
                    send_sem=y_ssem.at[c],
                    recv_sem=y_rsem.at[c],
                    device_id=yn,
                    device_id_type=pl.DeviceIdType.MESH,
                )
                rd.start()
                y_rdmas.append(rd)
            j = c - YLAG
            if 0 <= j < NZ:
                y_rdmas[j].wait_recv()
                rd = pltpu.make_async_remote_copy(
                    src_ref=out_ref.at[pl.ds(ot0 + rx(j), CH), :],
                    dst_ref=out_ref.at[pl.ds(ot0 + rx(j), CH), :],
                    send_sem=z2_ssem.at[j],
                    recv_sem=z2_rsem.at[j],
                    device_id=zn,
                    device_id_type=pl.DeviceIdType.MESH,
                )
                rd.start()
                z2_rdmas.append(rd)

        for i in range(NZ, NX):
            y_rdmas[i].wait_recv()
        for j in range(NZ):
            z2_rdmas[j].wait_recv()

        for i in range(NX):
            x_rdmas[i].wait_send()
            y_rdmas[i].wait_send()
        for j in range(NZ):
            z1_rdmas[j].wait_send()
            z2_rdmas[j].wait_send()

    return pl.pallas_call(
        body,
        out_shape=jax.ShapeDtypeStruct((m, d), jnp.bfloat16),
        in_specs=[
            pl.BlockSpec(memory_space=pltpu.VMEM),
            pl.BlockSpec(memory_space=pltpu.VMEM),
            pl.BlockSpec(memory_space=pltpu.VMEM),
        ],
        out_specs=pl.BlockSpec(memory_space=pltpu.VMEM),
        scratch_shapes=[
            pltpu.VMEM((NCH, CH, d), jnp.bfloat16),
            pltpu.SemaphoreType.DMA((NX,)),
            pltpu.SemaphoreType.DMA((NX,)),
            pltpu.SemaphoreType.DMA((NZ,)),
            pltpu.SemaphoreType.DMA((NZ,)),
            pltpu.SemaphoreType.DMA((NX,)),
            pltpu.SemaphoreType.DMA((NX,)),
            pltpu.SemaphoreType.DMA((NZ,)),
            pltpu.SemaphoreType.DMA((NZ,)),
        ],
        compiler_params=pltpu.CompilerParams(collective_id=0),
    )(p, resid, g)
